# baseline (speedup 1.0000x reference)
import functools

import jax
import jax.numpy as jnp
from jax import lax
from jax.experimental import pallas as pl
from jax.experimental.pallas import tpu as pltpu
from jax.experimental.pallas import tpu_sc as plsc

_NC = 2
_NS = 16
_NW = _NC * _NS
_L = 16


def _make_gather(N, M, D, nbuf):
    BB = 128
    assert N == BB * _NW and M % nbuf == 0

    def body(tokens_hbm, table_hbm, out_hbm, tok_v, hid_v, rows_v, tr_v, *sems):
        sem_g = sems[:nbuf]
        sem_s = sems[nbuf:]
        wid = lax.axis_index("s") * _NC + lax.axis_index("c")
        c0w = wid * BB

        iota = jnp.arange(_L, dtype=jnp.int32)
        rot = [(iota + d) & (_L - 1) for d in range(_L)]

        pltpu.sync_copy(tokens_hbm.at[:, pl.ds(c0w, BB)], tok_v)

        def transpose_block(b, s):
            def per_lgroup(lg, carry):
                l0 = lg * _L
                tokv = tok_v[s, pl.ds(l0, _L)]
                pv = (tokv & 1) << 6
                rowv = iota + l0
                for c0 in range(0, D, _L):
                    cp = pv + c0
                    for d in range(_L):
                        cl = cp + rot[d]
                        vals = plsc.load_gather(rows_v.at[b], [rowv, cl])
                        co = rot[d] + c0
                        plsc.store_scatter(tr_v.at[b], [co, rowv], vals)
                return carry

            lax.fori_loop(0, BB // _L, per_lgroup, 0)

        def group(g, carry):
            gathers = []
            for b in range(nbuf):
                s = g * nbuf + b

                @pl.when(g > 0)
                def _wait_prev():
                    pltpu.make_async_copy(
                        tr_v.at[b], out_hbm.at[0, :, pl.ds(c0w, BB)], sem_s[b]
                    ).wait()

                for k in range(BB // _L):
                    hid_v[b, pl.ds(k * _L, _L)] = (
                        tok_v[s, pl.ds(k * _L, _L)] >> 1
                    )
                gathers.append(
                    pltpu.async_copy(
                        table_hbm.at[hid_v.at[b]],
                        rows_v.at[b],
                        sem_g[b],
                    )
                )
            for b in range(nbuf):
                s = g * nbuf + b
                gathers[b].wait()
                transpose_block(b, s)
                pltpu.async_copy(
                    tr_v.at[b], out_hbm.at[s, :, pl.ds(c0w, BB)], sem_s[b]
                )
            return carry

        lax.fori_loop(0, M // nbuf, group, 0)

        for b in range(nbuf):
            pltpu.make_async_copy(
                tr_v.at[b], out_hbm.at[0, :, pl.ds(c0w, BB)], sem_s[b]
            ).wait()

    mesh = plsc.VectorSubcoreMesh(core_axis_name="c", subcore_axis_name="s")
    return pl.kernel(
        body,
        out_type=jax.ShapeDtypeStruct((M, D, N), jnp.float32),
        mesh=mesh,
        scratch_types=[
            pltpu.VMEM((M, BB), jnp.int32),
            pltpu.VMEM((nbuf, BB), jnp.int32),
            pltpu.VMEM((nbuf, BB, 2 * D), jnp.float32),
            pltpu.VMEM((nbuf, D, BB), jnp.float32),
        ]
        + [pltpu.SemaphoreType.DMA] * (2 * nbuf),
        compiler_params=pltpu.CompilerParams(
            use_tc_tiling_on_sc=True, needs_layout_passes=False
        ),
    )


def kernel(tokens, table):
    n, m = tokens.shape
    vocab, dim = table.shape
    tokens_t = tokens.T.astype(jnp.int32)
    table_pair = table.reshape(vocab // 2, 2 * dim)
    out = _make_gather(n, m, dim, 4)(tokens_t, table_pair)
    return jnp.transpose(out, (2, 0, 1))
